# Initial kernel scaffold; baseline (speedup 1.0000x reference)
import functools

import jax
import jax.numpy as jnp
from jax.experimental import pallas as pl
from jax.experimental.pallas import tpu as pltpu

NLIG = 50000
NKP = 5000
H = 64
V = 16

E_BLK = 1600
N_BLK = 2000
K_BLK = 1000

_S0, _S1 = 0, 64
_V0, _V1 = 64, 112
_X0, _X1 = 112, 115
_ROW = 128


def _silu(x):
    return x * jax.nn.sigmoid(x)


def _gvp_block(s, mv, whu, wf, b, v_out, gate):
    E = s.shape[0]
    h = whu.shape[1] - v_out
    vhu = jnp.dot(mv, whu, preferred_element_type=jnp.float32)

    def csum(a):
        return a[0:E] + a[E:2 * E] + a[2 * E:3 * E]

    sh = jnp.sqrt(csum(vhu[:, :h] * vhu[:, :h]) + 1e-8)
    s_out = _silu(jnp.dot(jnp.concatenate([s, sh], axis=1), wf,
                          preferred_element_type=jnp.float32) + b)
    vu = vhu[:, h:]
    if gate:
        n = jnp.sqrt(csum(vu * vu) + 1e-8)
        g = jax.nn.sigmoid(n)
        vu = jnp.concatenate([g, g, g], axis=0) * vu
    return s_out, vu


def _msg_body(src_ref, dst_ref, w1hu, w1f, b1, w2hu, w2f, b2, w3hu, w3f, b3,
              out_ref):
    src = src_ref[...]
    dst = dst_ref[...]
    E = src.shape[0]
    diff = dst[:, _X0:_X1] - src[:, _X0:_X1]
    nrm = jnp.sqrt(jnp.sum(diff * diff, axis=1, keepdims=True))
    d = diff / (nrm + 1e-8)
    mv = jnp.concatenate([
        jnp.concatenate([src[:, 64:80], dst[:, 64:80], d[:, 0:1]], axis=1),
        jnp.concatenate([src[:, 80:96], dst[:, 80:96], d[:, 1:2]], axis=1),
        jnp.concatenate([src[:, 96:112], dst[:, 96:112], d[:, 2:3]], axis=1),
    ], axis=0)
    s = jnp.concatenate([src[:, _S0:_S1], dst[:, _S0:_S1]], axis=1)
    s, v = _gvp_block(s, mv, w1hu[...], w1f[...], b1[...], V, True)
    s, v = _gvp_block(s, v, w2hu[...], w2f[...], b2[...], V, True)
    s, v = _gvp_block(s, v, w3hu[...], w3f[...], b3[...], V, True)
    out_ref[...] = jnp.concatenate(
        [s, v[0:E], v[E:2 * E], v[2 * E:3 * E],
         jnp.zeros((E, _ROW - _V1), jnp.float32)], axis=1)


def _ln(x, g, b):
    mu = jnp.mean(x, axis=-1, keepdims=True)
    var = jnp.mean((x - mu) * (x - mu), axis=-1, keepdims=True)
    return (x - mu) / jnp.sqrt(var + 1e-5) * g + b


def _upd_body(tab_ref, agg_ref, u1hu, u1f, ub1, u2hu, u2f, ub2, lng, lnb,
              out_ref):
    tab = tab_ref[...]
    agg = agg_ref[...]
    E = tab.shape[0]
    s0 = tab[:, _S0:_S1]
    mv = jnp.concatenate([
        jnp.concatenate([tab[:, 64:80], agg[:, 64:80]], axis=1),
        jnp.concatenate([tab[:, 80:96], agg[:, 80:96]], axis=1),
        jnp.concatenate([tab[:, 96:112], agg[:, 96:112]], axis=1),
    ], axis=0)
    s = jnp.concatenate([s0, agg[:, _S0:_S1]], axis=1)
    s, v = _gvp_block(s, mv, u1hu[...], u1f[...], ub1[...], V, True)
    s, v = _gvp_block(s, v, u2hu[...], u2f[...], ub2[...], V, True)
    s_new = _ln(s0 + s, lng[...], lnb[...])
    out_ref[...] = jnp.concatenate(
        [s_new,
         tab[:, 64:80] + v[0:E],
         tab[:, 80:96] + v[E:2 * E],
         tab[:, 96:112] + v[2 * E:3 * E],
         tab[:, _X0:_X1],
         jnp.zeros((E, _ROW - _X1), jnp.float32)], axis=1)


def _enc_body(inp_ref, w, b, lng, lnb, out_ref, *, has_v):
    inp = inp_ref[...]
    E = inp.shape[0]
    s = _ln(_silu(jnp.dot(inp[:, 0:65], w[...],
                          preferred_element_type=jnp.float32) + b[...]),
            lng[...], lnb[...])
    if has_v:
        vpart = inp[:, 68:116]
    else:
        vpart = jnp.zeros((E, 48), jnp.float32)
    out_ref[...] = jnp.concatenate(
        [s, vpart, inp[:, 65:68], jnp.zeros((E, _ROW - _X1), jnp.float32)],
        axis=1)


def _noise_body(tab_ref, n1hu, n1f, nb1, n2hu, n2f, nb2, n3hu, n3f, nb3,
                ow, ob, out_ref):
    tab = tab_ref[...]
    E = tab.shape[0]
    s = tab[:, _S0:_S1]
    mv = jnp.concatenate(
        [tab[:, 64:80], tab[:, 80:96], tab[:, 96:112]], axis=0)
    s, v = _gvp_block(s, mv, n1hu[...], n1f[...], nb1[...], V, True)
    s, v = _gvp_block(s, v, n2hu[...], n2f[...], nb2[...], V, True)
    s, v = _gvp_block(s, v, n3hu[...], n3f[...], nb3[...], 1, False)
    eps = jnp.dot(s, ow[...], preferred_element_type=jnp.float32) + ob[...]
    out_ref[...] = jnp.concatenate(
        [eps, v[0:E], v[E:2 * E], v[2 * E:3 * E],
         jnp.zeros((E, _ROW - 67), jnp.float32)], axis=1)


def _bcast(shape):
    return pl.BlockSpec(shape, lambda i: (0,) * len(shape))


def _whu(p):
    return jnp.concatenate([p['Wh'], p['Wh'] @ p['Wu']], axis=1)


def _gvp_args(p):
    return (_whu(p), p['Wf'], p['bf'].reshape(1, -1))


def _edge_messages(src_rows, dst_rows, chain):
    E = src_rows.shape[0]
    ws = _gvp_args(chain[0]) + _gvp_args(chain[1]) + _gvp_args(chain[2])
    return pl.pallas_call(
        _msg_body,
        grid=(E // E_BLK,),
        in_specs=[pl.BlockSpec((E_BLK, _ROW), lambda i: (i, 0)),
                  pl.BlockSpec((E_BLK, _ROW), lambda i: (i, 0))]
                 + [_bcast(w.shape) for w in ws],
        out_specs=pl.BlockSpec((E_BLK, _ROW), lambda i: (i, 0)),
        out_shape=jax.ShapeDtypeStruct((E, _ROW), jnp.float32),
    )(src_rows, dst_rows, *ws)


def _update(tab, agg, lp):
    ws = (_gvp_args(lp['upd'][0]) + _gvp_args(lp['upd'][1])
          + (lp['ln_g'].reshape(1, -1), lp['ln_b'].reshape(1, -1)))
    return pl.pallas_call(
        _upd_body,
        grid=(NLIG // N_BLK,),
        in_specs=[pl.BlockSpec((N_BLK, _ROW), lambda i: (i, 0)),
                  pl.BlockSpec((N_BLK, _ROW), lambda i: (i, 0))]
                 + [_bcast(w.shape) for w in ws],
        out_specs=pl.BlockSpec((N_BLK, _ROW), lambda i: (i, 0)),
        out_shape=jax.ShapeDtypeStruct((NLIG, _ROW), jnp.float32),
    )(tab, agg, *ws)


def _encode(inp, w, b, lng, lnb, blk, has_v):
    n, c = inp.shape
    ws = (w, b.reshape(1, -1), lng.reshape(1, -1), lnb.reshape(1, -1))
    return pl.pallas_call(
        functools.partial(_enc_body, has_v=has_v),
        grid=(n // blk,),
        in_specs=[pl.BlockSpec((blk, c), lambda i: (i, 0))]
                 + [_bcast(x.shape) for x in ws],
        out_specs=pl.BlockSpec((blk, _ROW), lambda i: (i, 0)),
        out_shape=jax.ShapeDtypeStruct((n, _ROW), jnp.float32),
    )(inp, *ws)


def _noise_head(tab, noise, ow, ob):
    ws = (_gvp_args(noise[0]) + _gvp_args(noise[1]) + _gvp_args(noise[2])
          + (ow, ob.reshape(1, -1)))
    return pl.pallas_call(
        _noise_body,
        grid=(NLIG // N_BLK,),
        in_specs=[pl.BlockSpec((N_BLK, _ROW), lambda i: (i, 0))]
                 + [_bcast(w.shape) for w in ws],
        out_specs=pl.BlockSpec((N_BLK, _ROW), lambda i: (i, 0)),
        out_shape=jax.ShapeDtypeStruct((NLIG, _ROW), jnp.float32),
    )(tab, *ws)


def kernel(lig_h0, lig_x0, kp_h0, kp_x0, kp_v0, timestep, lig_batch_idx,
           kp_batch_idx, ll_edge_index, kl_src, kl_dst, params):
    f32 = jnp.float32
    t_lig = timestep[lig_batch_idx][:, None].astype(f32)
    t_kp = timestep[kp_batch_idx][:, None].astype(f32)
    enc_lig = jnp.concatenate(
        [lig_h0, t_lig, lig_x0, jnp.zeros((NLIG, 4), f32)], axis=1)
    kp_vcm = jnp.transpose(kp_v0, (0, 2, 1)).reshape(NKP, 48)
    enc_kp = jnp.concatenate([kp_h0, t_kp, kp_x0, kp_vcm], axis=1)

    lig_tab = _encode(enc_lig, params['lig_enc_W'], params['lig_enc_b'],
                      params['lig_ln_g'], params['lig_ln_b'], N_BLK, False)
    kp_tab = _encode(enc_kp, params['kp_enc_W'], params['kp_enc_b'],
                     params['kp_ln_g'], params['kp_ln_b'], K_BLK, True)

    src = ll_edge_index[0]
    dst = ll_edge_index[1]
    for lp in params['convs']:
        msg_ll = _edge_messages(jnp.take(lig_tab, src, axis=0),
                                jnp.take(lig_tab, dst, axis=0), lp['ll_msg'])
        msg_kl = _edge_messages(jnp.take(kp_tab, kl_src, axis=0),
                                jnp.take(lig_tab, kl_dst, axis=0),
                                lp['kl_msg'])
        agg = (jax.ops.segment_sum(msg_ll, dst, num_segments=NLIG)
               + jax.ops.segment_sum(msg_kl, kl_dst, num_segments=NLIG))
        lig_tab = _update(lig_tab, agg, lp)

    out = _noise_head(lig_tab, params['noise'], params['out_W'],
                      params['out_b'])
    eps_h = out[:, 0:64]
    v = out[:, 64:67].reshape(NLIG, 1, 3)
    return eps_h, v


# speedup vs baseline: 9.9846x; 9.9846x over previous
import functools

import jax
import jax.numpy as jnp
from jax.experimental import pallas as pl
from jax.experimental.pallas import tpu as pltpu

NLIG = 50000
NKP = 5000
H = 64
V = 16

E_BLK = 1600
N_BLK = 2000
K_BLK = 1000

_S0, _S1 = 0, 64
_V0, _V1 = 64, 112
_X0, _X1 = 112, 115
_ROW = 128


def _silu(x):
    return x * jax.nn.sigmoid(x)


def _gvp_block(s, mv, wh, wu, wf, b, gate):
    E = s.shape[0]

    def csum(a):
        return a[0:E] + a[E:2 * E] + a[2 * E:3 * E]

    vh = jnp.dot(mv, wh, preferred_element_type=jnp.float32)
    sh = jnp.sqrt(csum(vh * vh) + 1e-8)
    s_out = _silu(jnp.dot(jnp.concatenate([s, sh], axis=1), wf,
                          preferred_element_type=jnp.float32) + b)
    vu = jnp.dot(vh, wu, preferred_element_type=jnp.float32)
    if gate:
        n = jnp.sqrt(csum(vu * vu) + 1e-8)
        g = jax.nn.sigmoid(n)
        vu = jnp.concatenate([g, g, g], axis=0) * vu
    return s_out, vu


def _msg_body(src_ref, dst_ref, w1h, w1u, w1f, b1, w2h, w2u, w2f, b2,
              w3h, w3u, w3f, b3, out_ref):
    src = src_ref[...]
    dst = dst_ref[...]
    E = src.shape[0]
    diff = dst[:, _X0:_X1] - src[:, _X0:_X1]
    nrm = jnp.sqrt(jnp.sum(diff * diff, axis=1, keepdims=True))
    d = diff / (nrm + 1e-8)
    mv = jnp.concatenate([
        jnp.concatenate([src[:, 64:80], dst[:, 64:80], d[:, 0:1]], axis=1),
        jnp.concatenate([src[:, 80:96], dst[:, 80:96], d[:, 1:2]], axis=1),
        jnp.concatenate([src[:, 96:112], dst[:, 96:112], d[:, 2:3]], axis=1),
    ], axis=0)
    s = jnp.concatenate([src[:, _S0:_S1], dst[:, _S0:_S1]], axis=1)
    s, v = _gvp_block(s, mv, w1h[...], w1u[...], w1f[...], b1[...], True)
    s, v = _gvp_block(s, v, w2h[...], w2u[...], w2f[...], b2[...], True)
    s, v = _gvp_block(s, v, w3h[...], w3u[...], w3f[...], b3[...], True)
    out_ref[...] = jnp.concatenate(
        [s, v[0:E], v[E:2 * E], v[2 * E:3 * E],
         jnp.zeros((E, _ROW - _V1), jnp.float32)], axis=1)


def _ln(x, g, b):
    mu = jnp.mean(x, axis=-1, keepdims=True)
    var = jnp.mean((x - mu) * (x - mu), axis=-1, keepdims=True)
    return (x - mu) / jnp.sqrt(var + 1e-5) * g + b


def _upd_body(tab_ref, agg_ref, u1h, u1u, u1f, ub1, u2h, u2u, u2f, ub2,
              lng, lnb, out_ref):
    tab = tab_ref[...]
    agg = agg_ref[...]
    E = tab.shape[0]
    s0 = tab[:, _S0:_S1]
    mv = jnp.concatenate([
        jnp.concatenate([tab[:, 64:80], agg[:, 64:80]], axis=1),
        jnp.concatenate([tab[:, 80:96], agg[:, 80:96]], axis=1),
        jnp.concatenate([tab[:, 96:112], agg[:, 96:112]], axis=1),
    ], axis=0)
    s = jnp.concatenate([s0, agg[:, _S0:_S1]], axis=1)
    s, v = _gvp_block(s, mv, u1h[...], u1u[...], u1f[...], ub1[...], True)
    s, v = _gvp_block(s, v, u2h[...], u2u[...], u2f[...], ub2[...], True)
    s_new = _ln(s0 + s, lng[...], lnb[...])
    out_ref[...] = jnp.concatenate(
        [s_new,
         tab[:, 64:80] + v[0:E],
         tab[:, 80:96] + v[E:2 * E],
         tab[:, 96:112] + v[2 * E:3 * E],
         tab[:, _X0:_X1],
         jnp.zeros((E, _ROW - _X1), jnp.float32)], axis=1)


def _enc_body(inp_ref, w, b, lng, lnb, out_ref, *, has_v):
    inp = inp_ref[...]
    E = inp.shape[0]
    s = _ln(_silu(jnp.dot(inp[:, 0:65], w[...],
                          preferred_element_type=jnp.float32) + b[...]),
            lng[...], lnb[...])
    if has_v:
        vpart = inp[:, 68:116]
    else:
        vpart = jnp.zeros((E, 48), jnp.float32)
    out_ref[...] = jnp.concatenate(
        [s, vpart, inp[:, 65:68], jnp.zeros((E, _ROW - _X1), jnp.float32)],
        axis=1)


def _noise_body(tab_ref, n1h, n1u, n1f, nb1, n2h, n2u, n2f, nb2,
                n3h, n3u, n3f, nb3, ow, ob, out_ref):
    tab = tab_ref[...]
    E = tab.shape[0]
    s = tab[:, _S0:_S1]
    mv = jnp.concatenate(
        [tab[:, 64:80], tab[:, 80:96], tab[:, 96:112]], axis=0)
    s, v = _gvp_block(s, mv, n1h[...], n1u[...], n1f[...], nb1[...], True)
    s, v = _gvp_block(s, v, n2h[...], n2u[...], n2f[...], nb2[...], True)
    s, v = _gvp_block(s, v, n3h[...], n3u[...], n3f[...], nb3[...], False)
    eps = jnp.dot(s, ow[...], preferred_element_type=jnp.float32) + ob[...]
    out_ref[...] = jnp.concatenate(
        [eps, v[0:E], v[E:2 * E], v[2 * E:3 * E],
         jnp.zeros((E, _ROW - 67), jnp.float32)], axis=1)


def _bcast(shape):
    return pl.BlockSpec(shape, lambda i: (0,) * len(shape))


def _gvp_args(p):
    return (p['Wh'], p['Wu'], p['Wf'], p['bf'].reshape(1, -1))


def _edge_messages(src_rows, dst_rows, chain):
    E = src_rows.shape[0]
    ws = _gvp_args(chain[0]) + _gvp_args(chain[1]) + _gvp_args(chain[2])
    return pl.pallas_call(
        _msg_body,
        grid=(E // E_BLK,),
        in_specs=[pl.BlockSpec((E_BLK, _ROW), lambda i: (i, 0)),
                  pl.BlockSpec((E_BLK, _ROW), lambda i: (i, 0))]
                 + [_bcast(w.shape) for w in ws],
        out_specs=pl.BlockSpec((E_BLK, _ROW), lambda i: (i, 0)),
        out_shape=jax.ShapeDtypeStruct((E, _ROW), jnp.float32),
    )(src_rows, dst_rows, *ws)


def _update(tab, agg, lp):
    ws = (_gvp_args(lp['upd'][0]) + _gvp_args(lp['upd'][1])
          + (lp['ln_g'].reshape(1, -1), lp['ln_b'].reshape(1, -1)))
    return pl.pallas_call(
        _upd_body,
        grid=(NLIG // N_BLK,),
        in_specs=[pl.BlockSpec((N_BLK, _ROW), lambda i: (i, 0)),
                  pl.BlockSpec((N_BLK, _ROW), lambda i: (i, 0))]
                 + [_bcast(w.shape) for w in ws],
        out_specs=pl.BlockSpec((N_BLK, _ROW), lambda i: (i, 0)),
        out_shape=jax.ShapeDtypeStruct((NLIG, _ROW), jnp.float32),
    )(tab, agg, *ws)


def _encode(inp, w, b, lng, lnb, blk, has_v):
    n, c = inp.shape
    ws = (w, b.reshape(1, -1), lng.reshape(1, -1), lnb.reshape(1, -1))
    return pl.pallas_call(
        functools.partial(_enc_body, has_v=has_v),
        grid=(n // blk,),
        in_specs=[pl.BlockSpec((blk, c), lambda i: (i, 0))]
                 + [_bcast(x.shape) for x in ws],
        out_specs=pl.BlockSpec((blk, _ROW), lambda i: (i, 0)),
        out_shape=jax.ShapeDtypeStruct((n, _ROW), jnp.float32),
    )(inp, *ws)


def _noise_head(tab, noise, ow, ob):
    ws = (_gvp_args(noise[0]) + _gvp_args(noise[1]) + _gvp_args(noise[2])
          + (ow, ob.reshape(1, -1)))
    return pl.pallas_call(
        _noise_body,
        grid=(NLIG // N_BLK,),
        in_specs=[pl.BlockSpec((N_BLK, _ROW), lambda i: (i, 0))]
                 + [_bcast(w.shape) for w in ws],
        out_specs=pl.BlockSpec((N_BLK, _ROW), lambda i: (i, 0)),
        out_shape=jax.ShapeDtypeStruct((NLIG, _ROW), jnp.float32),
    )(tab, *ws)


def kernel(lig_h0, lig_x0, kp_h0, kp_x0, kp_v0, timestep, lig_batch_idx,
           kp_batch_idx, ll_edge_index, kl_src, kl_dst, params):
    f32 = jnp.float32
    t_lig = timestep[lig_batch_idx][:, None].astype(f32)
    t_kp = timestep[kp_batch_idx][:, None].astype(f32)
    enc_lig = jnp.concatenate(
        [lig_h0, t_lig, lig_x0, jnp.zeros((NLIG, 4), f32)], axis=1)
    kp_vcm = jnp.transpose(kp_v0, (0, 2, 1)).reshape(NKP, 48)
    enc_kp = jnp.concatenate([kp_h0, t_kp, kp_x0, kp_vcm], axis=1)

    lig_tab = _encode(enc_lig, params['lig_enc_W'], params['lig_enc_b'],
                      params['lig_ln_g'], params['lig_ln_b'], N_BLK, False)
    kp_tab = _encode(enc_kp, params['kp_enc_W'], params['kp_enc_b'],
                     params['kp_ln_g'], params['kp_ln_b'], K_BLK, True)

    src = ll_edge_index[0]
    dst = ll_edge_index[1]
    for lp in params['convs']:
        msg_ll = _edge_messages(jnp.take(lig_tab, src, axis=0),
                                jnp.take(lig_tab, dst, axis=0), lp['ll_msg'])
        msg_kl = _edge_messages(jnp.take(kp_tab, kl_src, axis=0),
                                jnp.take(lig_tab, kl_dst, axis=0),
                                lp['kl_msg'])
        agg = (jax.ops.segment_sum(msg_ll, dst, num_segments=NLIG)
               + jax.ops.segment_sum(msg_kl, kl_dst, num_segments=NLIG))
        lig_tab = _update(lig_tab, agg, lp)

    out = _noise_head(lig_tab, params['noise'], params['out_W'],
                      params['out_b'])
    eps_h = out[:, 0:64]
    v = out[:, 64:67].reshape(NLIG, 1, 3)
    return eps_h, v
